# Initial kernel scaffold; baseline (speedup 1.0000x reference)
import jax
import jax.numpy as jnp
from jax.experimental import pallas as pl


def kernel(inputs, embed_weight):
    raise NotImplementedError("write your pallas kernel here")


# speedup vs baseline: 11.3475x; 11.3475x over previous
import functools

import jax
import jax.numpy as jnp
from jax import lax
from jax.experimental import pallas as pl
from jax.experimental.pallas import tpu as pltpu
from jax.experimental.pallas import tpu_sc as plsc

_K = 8192
_D = 32
_N = 16384
_BETA = 0.25

_ROWS = 512
_CHUNK = 2048
_NCHUNK = _K // _CHUNK
_NBLK = _N // _ROWS


def _argmin_body(z_ref, et_ref, s_ref, e2_ref, idx_ref, loss_ref):
    i = pl.program_id(0)
    z = z_ref[...]
    s = s_ref[...]
    half_min = [None, None]
    half_idx = [None, None]
    for c in range(_NCHUNK):
        et = et_ref[:, c * _CHUNK:(c + 1) * _CHUNK]
        e2 = e2_ref[:, c * _CHUNK:(c + 1) * _CHUNK]
        m = jnp.dot(z, et, preferred_element_type=jnp.float32)
        d = (s + e2) - 2.0 * m
        cmin = jnp.min(d, axis=1, keepdims=True)
        gidx = lax.broadcasted_iota(jnp.int32, (_ROWS, _CHUNK), 1) + c * _CHUNK
        cidx = jnp.min(jnp.where(d == cmin, gidx, _K), axis=1, keepdims=True)
        h = (c * _CHUNK) // (_K // 2)
        if half_min[h] is None:
            half_min[h], half_idx[h] = cmin, cidx
        else:
            better = cmin < half_min[h]
            half_min[h] = jnp.where(better, cmin, half_min[h])
            half_idx[h] = jnp.where(better, cidx, half_idx[h])
    m0, m1 = half_min
    h0, h1 = half_idx
    b0 = m0.astype(jnp.bfloat16).astype(jnp.float32)
    sel = m1 < b0
    idx_ref[...] = jnp.where(sel, h1, h0)

    @pl.when(i == 0)
    def _():
        loss_ref[...] = jnp.zeros((1, 1), jnp.float32)

    loss_ref[...] += jnp.sum(jnp.where(sel, m1, m0), keepdims=True)


def _tc_argmin(z, et, s, e2):
    return pl.pallas_call(
        _argmin_body,
        grid=(_NBLK,),
        in_specs=[
            pl.BlockSpec((_ROWS, _D), lambda i: (i, 0)),
            pl.BlockSpec((_D, _K), lambda i: (0, 0)),
            pl.BlockSpec((_ROWS, 1), lambda i: (i, 0)),
            pl.BlockSpec((1, _K), lambda i: (0, 0)),
        ],
        out_specs=[
            pl.BlockSpec((_ROWS, 1), lambda i: (i, 0)),
            pl.BlockSpec((1, 1), lambda i: (0, 0)),
        ],
        out_shape=[
            jax.ShapeDtypeStruct((_N, 1), jnp.int32),
            jax.ShapeDtypeStruct((1, 1), jnp.float32),
        ],
    )(z, et, s, e2)


_DPAD = 128


def _make_sc_gather():
    info = plsc.get_sparse_core_info()
    nw = info.num_cores * info.num_subcores
    b_per_w = _N // nw
    mesh = plsc.VectorSubcoreMesh(core_axis_name="c", subcore_axis_name="s")

    @functools.partial(
        pl.kernel, mesh=mesh,
        out_type=jax.ShapeDtypeStruct((_N, _DPAD), jnp.float32),
        scratch_types=[
            pltpu.VMEM((b_per_w,), jnp.int32),
            pltpu.VMEM((b_per_w, _DPAD), jnp.float32),
            pltpu.SemaphoreType.DMA,
        ],
    )
    def gather_rows(table_hbm, idx_hbm, out_hbm, idx_v, rows_v, sem):
        wid = lax.axis_index("s") * info.num_cores + lax.axis_index("c")
        base = wid * b_per_w
        pltpu.sync_copy(idx_hbm.at[pl.ds(base, b_per_w)], idx_v)
        pltpu.async_copy(table_hbm.at[idx_v], rows_v, sem).wait()
        pltpu.sync_copy(rows_v, out_hbm.at[pl.ds(base, b_per_w)])

    return gather_rows


def kernel(inputs, embed_weight):
    flat = inputs.reshape(-1, _D)
    s = jnp.sum(flat ** 2, axis=1, keepdims=True)
    e2 = jnp.sum(embed_weight ** 2, axis=1)
    idx2d, loss_sum = _tc_argmin(flat, embed_weight.T, s, e2.reshape(1, _K))

    table = jnp.pad(embed_weight, ((0, 0), (0, _DPAD - _D)))
    gathered = _make_sc_gather()(table, idx2d.reshape(_N))[:, :_D]

    mean_sq = loss_sum[0, 0] / (_N * _D)
    commitloss = mean_sq + _BETA * mean_sq
    quantized = flat + lax.stop_gradient(gathered - flat)
    return (quantized.reshape(inputs.shape), idx2d, commitloss,
            jnp.array(0.0, dtype=jnp.float32))


# speedup vs baseline: 12.7583x; 1.1243x over previous
import functools

import jax
import jax.numpy as jnp
from jax import lax
from jax.experimental import pallas as pl
from jax.experimental.pallas import tpu as pltpu
from jax.experimental.pallas import tpu_sc as plsc

_K = 8192
_D = 32
_N = 16384
_BETA = 0.25

_ROWS = 512
_CHUNK = 2048
_NCHUNK = _K // _CHUNK
_NBLK = _N // _ROWS


def _argmin_body(z2_ref, et_ref, s_ref, e2_ref, idx_ref, loss_ref):
    i = pl.program_id(0)
    z2 = z2_ref[...]
    s = s_ref[...]
    half_min = [None, None]
    half_idx = [None, None]
    for c in range(_NCHUNK):
        et = et_ref[:, c * _CHUNK:(c + 1) * _CHUNK]
        e2 = e2_ref[:, c * _CHUNK:(c + 1) * _CHUNK]
        m2 = jnp.dot(z2, et, preferred_element_type=jnp.float32)
        d = (s + e2) - m2
        cmin = jnp.min(d, axis=1, keepdims=True)
        gidx = lax.broadcasted_iota(
            jnp.int32, (_ROWS, _CHUNK), 1).astype(jnp.float32)
        cidx = jnp.min(jnp.where(d == cmin, gidx, float(_K)),
                       axis=1, keepdims=True) + float(c * _CHUNK)
        h = (c * _CHUNK) // (_K // 2)
        if half_min[h] is None:
            half_min[h], half_idx[h] = cmin, cidx
        else:
            better = cmin < half_min[h]
            half_min[h] = jnp.where(better, cmin, half_min[h])
            half_idx[h] = jnp.where(better, cidx, half_idx[h])
    m0, m1 = half_min
    h0, h1 = half_idx
    b0 = m0.astype(jnp.bfloat16).astype(jnp.float32)
    sel = m1 < b0
    idx_ref[...] = jnp.where(sel, h1, h0).astype(jnp.int32)

    @pl.when(i == 0)
    def _():
        loss_ref[...] = jnp.zeros((1, 1), jnp.float32)

    loss_ref[...] += jnp.sum(jnp.where(sel, m1, m0), keepdims=True)


def _tc_argmin(z2, et, s, e2):
    return pl.pallas_call(
        _argmin_body,
        grid=(_NBLK,),
        in_specs=[
            pl.BlockSpec((_ROWS, _D), lambda i: (i, 0)),
            pl.BlockSpec((_D, _K), lambda i: (0, 0)),
            pl.BlockSpec((_ROWS, 1), lambda i: (i, 0)),
            pl.BlockSpec((1, _K), lambda i: (0, 0)),
        ],
        out_specs=[
            pl.BlockSpec((_ROWS, 1), lambda i: (i, 0)),
            pl.BlockSpec((1, 1), lambda i: (0, 0)),
        ],
        out_shape=[
            jax.ShapeDtypeStruct((_N, 1), jnp.int32),
            jax.ShapeDtypeStruct((1, 1), jnp.float32),
        ],
    )(z2, et, s, e2)


_DPAD = 128


def _make_sc_gather():
    info = plsc.get_sparse_core_info()
    nw = info.num_cores * info.num_subcores
    b_per_w = _N // nw
    mesh = plsc.VectorSubcoreMesh(core_axis_name="c", subcore_axis_name="s")

    @functools.partial(
        pl.kernel, mesh=mesh,
        out_type=jax.ShapeDtypeStruct((_N, _DPAD), jnp.float32),
        scratch_types=[
            pltpu.VMEM((b_per_w,), jnp.int32),
            pltpu.VMEM((b_per_w, _DPAD), jnp.float32),
            pltpu.SemaphoreType.DMA,
        ],
    )
    def gather_rows(table_hbm, idx_hbm, out_hbm, idx_v, rows_v, sem):
        wid = lax.axis_index("s") * info.num_cores + lax.axis_index("c")
        base = wid * b_per_w
        pltpu.sync_copy(idx_hbm.at[pl.ds(base, b_per_w)], idx_v)
        pltpu.async_copy(table_hbm.at[idx_v], rows_v, sem).wait()
        pltpu.sync_copy(rows_v, out_hbm.at[pl.ds(base, b_per_w)])

    return gather_rows


def kernel(inputs, embed_weight):
    flat = inputs.reshape(-1, _D)
    s = jnp.sum(flat ** 2, axis=1, keepdims=True)
    e2 = jnp.sum(embed_weight ** 2, axis=1)
    idx2d, loss_sum = _tc_argmin(2.0 * flat, embed_weight.T, s,
                                 e2.reshape(1, _K))

    table = jnp.pad(embed_weight, ((0, 0), (0, _DPAD - _D)))
    gathered = _make_sc_gather()(table, idx2d.reshape(_N))[:, :_D]

    mean_sq = loss_sum[0, 0] / (_N * _D)
    commitloss = mean_sq + _BETA * mean_sq
    quantized = flat + lax.stop_gradient(gathered - flat)
    return (quantized.reshape(inputs.shape), idx2d, commitloss,
            jnp.array(0.0, dtype=jnp.float32))


# speedup vs baseline: 14.7398x; 1.1553x over previous
import functools

import jax
import jax.numpy as jnp
from jax import lax
from jax.experimental import pallas as pl
from jax.experimental.pallas import tpu as pltpu
from jax.experimental.pallas import tpu_sc as plsc

_K = 8192
_D = 32
_N = 16384
_BETA = 0.25

_ROWS = 512
_CHUNK = 2048
_NCHUNK = _K // _CHUNK
_NBLK = _N // _ROWS


def _argmin_body(z2_ref, et_ref, s_ref, e2_ref, idx_ref, loss_ref):
    i = pl.program_id(0)
    z2 = z2_ref[...]
    s = s_ref[...]
    half_min = [None, None]
    half_idx = [None, None]
    lane = lax.broadcasted_iota(
        jnp.int32, (_ROWS, 128), 1).astype(jnp.float32)
    ntile = _CHUNK // 128
    for h in range(2):
        run_v = None
        run_t = None
        for ch in range(_NCHUNK // 2):
            c = h * (_NCHUNK // 2) + ch
            et = et_ref[:, c * _CHUNK:(c + 1) * _CHUNK]
            m2 = jnp.dot(z2, et, preferred_element_type=jnp.float32)
            for t in range(ntile):
                col = pl.ds(c * _CHUNK + t * 128, 128)
                dt = (s + e2_ref[:, col]) - m2[:, t * 128:(t + 1) * 128]
                if run_v is None:
                    run_v = dt
                    run_t = jnp.zeros((_ROWS, 128), jnp.float32)
                else:
                    upd = dt < run_v
                    run_v = jnp.where(upd, dt, run_v)
                    run_t = jnp.where(upd, float(ch * ntile + t), run_t)
        hmin = jnp.min(run_v, axis=1, keepdims=True)
        jlane = run_t * 128.0 + lane
        hidx = jnp.min(jnp.where(run_v == hmin, jlane, float(_K)),
                       axis=1, keepdims=True) + float(h * (_K // 2))
        half_min[h], half_idx[h] = hmin, hidx
    m0, m1 = half_min
    h0, h1 = half_idx
    b0 = m0.astype(jnp.bfloat16).astype(jnp.float32)
    sel = m1 < b0
    idx_ref[...] = jnp.where(sel, h1, h0).astype(jnp.int32)

    @pl.when(i == 0)
    def _():
        loss_ref[...] = jnp.zeros((1, 1), jnp.float32)

    loss_ref[...] += jnp.sum(jnp.where(sel, m1, m0), keepdims=True)


def _tc_argmin(z2, et, s, e2):
    return pl.pallas_call(
        _argmin_body,
        grid=(_NBLK,),
        in_specs=[
            pl.BlockSpec((_ROWS, _D), lambda i: (i, 0)),
            pl.BlockSpec((_D, _K), lambda i: (0, 0)),
            pl.BlockSpec((_ROWS, 1), lambda i: (i, 0)),
            pl.BlockSpec((1, _K), lambda i: (0, 0)),
        ],
        out_specs=[
            pl.BlockSpec((_ROWS, 1), lambda i: (i, 0)),
            pl.BlockSpec((1, 1), lambda i: (0, 0)),
        ],
        out_shape=[
            jax.ShapeDtypeStruct((_N, 1), jnp.int32),
            jax.ShapeDtypeStruct((1, 1), jnp.float32),
        ],
    )(z2, et, s, e2)


_DPAD = 128


def _make_sc_gather():
    info = plsc.get_sparse_core_info()
    nw = info.num_cores * info.num_subcores
    b_per_w = _N // nw
    mesh = plsc.VectorSubcoreMesh(core_axis_name="c", subcore_axis_name="s")

    @functools.partial(
        pl.kernel, mesh=mesh,
        out_type=jax.ShapeDtypeStruct((_N, _DPAD), jnp.float32),
        scratch_types=[
            pltpu.VMEM((b_per_w,), jnp.int32),
            pltpu.VMEM((b_per_w, _DPAD), jnp.float32),
            pltpu.SemaphoreType.DMA,
        ],
    )
    def gather_rows(table_hbm, idx_hbm, out_hbm, idx_v, rows_v, sem):
        wid = lax.axis_index("s") * info.num_cores + lax.axis_index("c")
        base = wid * b_per_w
        pltpu.sync_copy(idx_hbm.at[pl.ds(base, b_per_w)], idx_v)
        pltpu.async_copy(table_hbm.at[idx_v], rows_v, sem).wait()
        pltpu.sync_copy(rows_v, out_hbm.at[pl.ds(base, b_per_w)])

    return gather_rows


def kernel(inputs, embed_weight):
    flat = inputs.reshape(-1, _D)
    s = jnp.sum(flat ** 2, axis=1, keepdims=True)
    e2 = jnp.sum(embed_weight ** 2, axis=1)
    idx2d, loss_sum = _tc_argmin(2.0 * flat, embed_weight.T, s,
                                 e2.reshape(1, _K))

    table = jnp.pad(embed_weight, ((0, 0), (0, _DPAD - _D)))
    gathered = _make_sc_gather()(table, idx2d.reshape(_N))[:, :_D]

    mean_sq = loss_sum[0, 0] / (_N * _D)
    commitloss = mean_sq + _BETA * mean_sq
    quantized = flat + lax.stop_gradient(gathered - flat)
    return (quantized.reshape(inputs.shape), idx2d, commitloss,
            jnp.array(0.0, dtype=jnp.float32))


# speedup vs baseline: 14.7871x; 1.0032x over previous
import functools

import jax
import jax.numpy as jnp
from jax import lax
from jax.experimental import pallas as pl
from jax.experimental.pallas import tpu as pltpu
from jax.experimental.pallas import tpu_sc as plsc

_K = 8192
_D = 32
_N = 16384
_BETA = 0.25

_ROWS = 512
_CHUNK = 2048
_NCHUNK = _K // _CHUNK
_NBLK = _N // _ROWS


def _argmin_body(z_ref, e_ref, s_ref, e2_ref, idx_ref, loss_ref):
    i = pl.program_id(0)
    z = z_ref[...]
    z2 = z + z
    s = s_ref[...]
    half_min = [None, None]
    half_idx = [None, None]
    lane = lax.broadcasted_iota(
        jnp.int32, (_ROWS, 128), 1).astype(jnp.float32)
    ntile = _CHUNK // 128
    for h in range(2):
        run_v = None
        run_t = None
        for ch in range(_NCHUNK // 2):
            c = h * (_NCHUNK // 2) + ch
            ec = e_ref[pl.ds(c * _CHUNK, _CHUNK), :]
            m2 = lax.dot_general(z2, ec, (((1,), (1,)), ((), ())),
                                 preferred_element_type=jnp.float32)
            for t in range(ntile):
                col = pl.ds(c * _CHUNK + t * 128, 128)
                dt = (s + e2_ref[:, col]) - m2[:, t * 128:(t + 1) * 128]
                if run_v is None:
                    run_v = dt
                    run_t = jnp.zeros((_ROWS, 128), jnp.float32)
                else:
                    upd = dt < run_v
                    run_v = jnp.where(upd, dt, run_v)
                    run_t = jnp.where(upd, float(ch * ntile + t), run_t)
        hmin = jnp.min(run_v, axis=1, keepdims=True)
        jlane = run_t * 128.0 + lane
        hidx = jnp.min(jnp.where(run_v == hmin, jlane, float(_K)),
                       axis=1, keepdims=True) + float(h * (_K // 2))
        half_min[h], half_idx[h] = hmin, hidx
    m0, m1 = half_min
    h0, h1 = half_idx
    b0 = m0.astype(jnp.bfloat16).astype(jnp.float32)
    sel = m1 < b0
    idx_ref[...] = jnp.where(sel, h1, h0).astype(jnp.int32)

    @pl.when(i == 0)
    def _():
        loss_ref[...] = jnp.zeros((1, 1), jnp.float32)

    loss_ref[...] += jnp.sum(jnp.where(sel, m1, m0), keepdims=True)


def _tc_argmin(z, e, s, e2):
    return pl.pallas_call(
        _argmin_body,
        grid=(_NBLK,),
        in_specs=[
            pl.BlockSpec((_ROWS, _D), lambda i: (i, 0)),
            pl.BlockSpec((_K, _D), lambda i: (0, 0)),
            pl.BlockSpec((_ROWS, 1), lambda i: (i, 0)),
            pl.BlockSpec((1, _K), lambda i: (0, 0)),
        ],
        out_specs=[
            pl.BlockSpec((_ROWS, 1), lambda i: (i, 0)),
            pl.BlockSpec((1, 1), lambda i: (0, 0)),
        ],
        out_shape=[
            jax.ShapeDtypeStruct((_N, 1), jnp.int32),
            jax.ShapeDtypeStruct((1, 1), jnp.float32),
        ],
    )(z, e, s, e2)


_DPAD = 128


def _make_sc_gather():
    info = plsc.get_sparse_core_info()
    nw = info.num_cores * info.num_subcores
    b_per_w = _N // nw
    mesh = plsc.VectorSubcoreMesh(core_axis_name="c", subcore_axis_name="s")

    @functools.partial(
        pl.kernel, mesh=mesh,
        out_type=jax.ShapeDtypeStruct((_N, _DPAD), jnp.float32),
        scratch_types=[
            pltpu.VMEM((b_per_w,), jnp.int32),
            pltpu.VMEM((b_per_w, _DPAD), jnp.float32),
            pltpu.SemaphoreType.DMA,
        ],
    )
    def gather_rows(table_hbm, idx_hbm, out_hbm, idx_v, rows_v, sem):
        wid = lax.axis_index("s") * info.num_cores + lax.axis_index("c")
        base = wid * b_per_w
        pltpu.sync_copy(idx_hbm.at[pl.ds(base, b_per_w)], idx_v)
        pltpu.async_copy(table_hbm.at[idx_v], rows_v, sem).wait()
        pltpu.sync_copy(rows_v, out_hbm.at[pl.ds(base, b_per_w)])

    return gather_rows


def kernel(inputs, embed_weight):
    flat = inputs.reshape(-1, _D)
    s = jnp.sum(flat ** 2, axis=1, keepdims=True)
    e2 = jnp.sum(embed_weight ** 2, axis=1)
    idx2d, loss_sum = _tc_argmin(flat, embed_weight, s, e2.reshape(1, _K))

    table = jnp.pad(embed_weight, ((0, 0), (0, _DPAD - _D)))
    gathered = _make_sc_gather()(table, idx2d.reshape(_N))[:, :_D]

    mean_sq = loss_sum[0, 0] / (_N * _D)
    commitloss = mean_sq + _BETA * mean_sq
    quantized = flat + lax.stop_gradient(gathered - flat)
    return (quantized.reshape(inputs.shape), idx2d, commitloss,
            jnp.array(0.0, dtype=jnp.float32))


# speedup vs baseline: 15.5599x; 1.0523x over previous
import functools

import jax
import jax.numpy as jnp
from jax import lax
from jax.experimental import pallas as pl
from jax.experimental.pallas import tpu as pltpu
from jax.experimental.pallas import tpu_sc as plsc

_K = 8192
_D = 32
_N = 16384
_BETA = 0.25

_ROWS = 1024
_CHUNK = 2048
_NCHUNK = _K // _CHUNK
_NBLK = _N // _ROWS


def _argmin_body(z_ref, e_ref, s_ref, e2_ref, idx_ref, loss_ref):
    i = pl.program_id(0)
    z = z_ref[...]
    z2 = z + z
    s = s_ref[...]
    half_min = [None, None]
    half_idx = [None, None]
    lane = lax.broadcasted_iota(
        jnp.int32, (_ROWS, 128), 1).astype(jnp.float32)
    ntile = _CHUNK // 128
    for h in range(2):
        run_v = None
        run_t = None
        for ch in range(_NCHUNK // 2):
            c = h * (_NCHUNK // 2) + ch
            ec = e_ref[pl.ds(c * _CHUNK, _CHUNK), :]
            m2 = lax.dot_general(z2, ec, (((1,), (1,)), ((), ())),
                                 preferred_element_type=jnp.float32)
            for t in range(ntile):
                col = pl.ds(c * _CHUNK + t * 128, 128)
                dt = (s + e2_ref[:, col]) - m2[:, t * 128:(t + 1) * 128]
                if run_v is None:
                    run_v = dt
                    run_t = jnp.zeros((_ROWS, 128), jnp.float32)
                else:
                    upd = dt < run_v
                    run_v = jnp.where(upd, dt, run_v)
                    run_t = jnp.where(upd, float(ch * ntile + t), run_t)
        hmin = jnp.min(run_v, axis=1, keepdims=True)
        jlane = run_t * 128.0 + lane
        hidx = jnp.min(jnp.where(run_v == hmin, jlane, float(_K)),
                       axis=1, keepdims=True) + float(h * (_K // 2))
        half_min[h], half_idx[h] = hmin, hidx
    m0, m1 = half_min
    h0, h1 = half_idx
    b0 = m0.astype(jnp.bfloat16).astype(jnp.float32)
    sel = m1 < b0
    idx_ref[...] = jnp.where(sel, h1, h0).astype(jnp.int32)

    @pl.when(i == 0)
    def _():
        loss_ref[...] = jnp.zeros((1, 1), jnp.float32)

    loss_ref[...] += jnp.sum(jnp.where(sel, m1, m0), keepdims=True)


def _tc_argmin(z, e, s, e2):
    return pl.pallas_call(
        _argmin_body,
        grid=(_NBLK,),
        in_specs=[
            pl.BlockSpec((_ROWS, _D), lambda i: (i, 0)),
            pl.BlockSpec((_K, _D), lambda i: (0, 0)),
            pl.BlockSpec((_ROWS, 1), lambda i: (i, 0)),
            pl.BlockSpec((1, _K), lambda i: (0, 0)),
        ],
        out_specs=[
            pl.BlockSpec((_ROWS, 1), lambda i: (i, 0)),
            pl.BlockSpec((1, 1), lambda i: (0, 0)),
        ],
        out_shape=[
            jax.ShapeDtypeStruct((_N, 1), jnp.int32),
            jax.ShapeDtypeStruct((1, 1), jnp.float32),
        ],
    )(z, e, s, e2)


_DPAD = 128


def _make_sc_gather():
    info = plsc.get_sparse_core_info()
    nw = info.num_cores * info.num_subcores
    b_per_w = _N // nw
    mesh = plsc.VectorSubcoreMesh(core_axis_name="c", subcore_axis_name="s")

    @functools.partial(
        pl.kernel, mesh=mesh,
        out_type=jax.ShapeDtypeStruct((_N, _DPAD), jnp.float32),
        scratch_types=[
            pltpu.VMEM((b_per_w,), jnp.int32),
            pltpu.VMEM((b_per_w, _DPAD), jnp.float32),
            pltpu.SemaphoreType.DMA,
        ],
    )
    def gather_rows(table_hbm, idx_hbm, out_hbm, idx_v, rows_v, sem):
        wid = lax.axis_index("s") * info.num_cores + lax.axis_index("c")
        base = wid * b_per_w
        pltpu.sync_copy(idx_hbm.at[pl.ds(base, b_per_w)], idx_v)
        pltpu.async_copy(table_hbm.at[idx_v], rows_v, sem).wait()
        pltpu.sync_copy(rows_v, out_hbm.at[pl.ds(base, b_per_w)])

    return gather_rows


def kernel(inputs, embed_weight):
    flat = inputs.reshape(-1, _D)
    s = jnp.sum(flat ** 2, axis=1, keepdims=True)
    e2 = jnp.sum(embed_weight ** 2, axis=1)
    idx2d, loss_sum = _tc_argmin(flat, embed_weight, s, e2.reshape(1, _K))

    table = jnp.pad(embed_weight, ((0, 0), (0, _DPAD - _D)))
    gathered = _make_sc_gather()(table, idx2d.reshape(_N))[:, :_D]

    mean_sq = loss_sum[0, 0] / (_N * _D)
    commitloss = mean_sq + _BETA * mean_sq
    quantized = flat + lax.stop_gradient(gathered - flat)
    return (quantized.reshape(inputs.shape), idx2d, commitloss,
            jnp.array(0.0, dtype=jnp.float32))
